# baseline (speedup 1.0000x reference)
import jax
import jax.numpy as jnp
from jax.experimental import pallas as pl
from jax.experimental.pallas import tpu as pltpu

EPS = 1e-09
H = 2


def _fused_kernel(mask_ref, w_ref, wexp_ref, wbias_ref, bias_ref, x_ref,
                  o_ref, stats_ref):
    p = pl.program_id(1)
    h = pl.program_id(2)
    nf = x_ref.shape[1]
    n = x_ref.shape[2]
    fh = nf // H

    m = mask_ref[0]
    md = m * m
    lane = jax.lax.broadcasted_iota(jnp.int32, (1, n), 1)
    ii = jax.lax.broadcasted_iota(jnp.int32, (n, n), 0)
    kk = jax.lax.broadcasted_iota(jnp.int32, (n, n), 1)
    eye = ii == kk

    @pl.when(p == 0)
    def _stats():
        e2 = jnp.where(eye, md, 0.0)
        trvec = jnp.zeros((1, n), jnp.float32)
        qs = jnp.zeros((n, n), jnp.float32)
        for g in range(fh):
            fidx = h * fh + g
            a = x_ref[0, fidx]
            trvec = trvec + jnp.where(lane == fidx, jnp.sum(a * e2), 0.0)
            qs = qs + a * jnp.transpose(a)
        s2 = jnp.sum(qs * md)
        s2vec = jnp.where(lane == 0, s2, 0.0)

        @pl.when(h == 0)
        def _():
            stats_ref[0:1, :] = trvec
            stats_ref[1:2, :] = s2vec

        @pl.when(h != 0)
        def _():
            stats_ref[0:1, :] = stats_ref[0:1, :] + trvec
            stats_ref[1:2, :] = stats_ref[1:2, :] + s2vec

    @pl.when(p == 1)
    def _normalize():
        cnt = jnp.sum(md)
        cnt2 = jnp.maximum(cnt - 1.0, 1.0)
        trvec = stats_ref[0:1, :]
        s1 = jnp.sum(trvec)
        s3 = jnp.sum(trvec * trvec)
        s2 = jnp.sum(stats_ref[1:2, :])
        mean_b = s1 / (cnt * nf)
        var_b = (s2 - s3 / cnt) / (cnt2 * nf)
        inv = jax.lax.rsqrt(var_b + EPS)

        scale_vec = w_ref[...] * jnp.exp(wexp_ref[...]) + wbias_ref[...]
        flane = jax.lax.broadcasted_iota(jnp.int32, (1, nf), 1)
        e1 = jnp.where(eye, 1.0, 0.0)
        m2i = (jnp.transpose(m) * m) * inv
        me1 = mean_b * e1

        for g in range(fh):
            fidx = h * fh + g
            fsel = flane == fidx
            s = jnp.sum(jnp.where(fsel, scale_vec, 0.0))
            bf = jnp.sum(jnp.where(fsel, bias_ref[...], 0.0))
            a = x_ref[0, fidx]
            o_ref[0, g] = ((a - me1) * m2i) * s + bf * e1


def kernel(x, mask, weight, weight_exp, weight_bias, bias):
    b, f, n, _ = x.shape
    fh = f // H
    mask3 = mask.reshape(b, 1, n)
    w2 = weight.reshape(1, f)
    wexp2 = weight_exp.reshape(1, f)
    wb2 = weight_bias.reshape(1, f)
    bias2 = bias.reshape(1, f)

    out = pl.pallas_call(
        _fused_kernel,
        grid=(b, 2, H),
        in_specs=[
            pl.BlockSpec((1, 1, n), lambda i, p, h: (i, 0, 0)),
            pl.BlockSpec((1, f), lambda i, p, h: (0, 0)),
            pl.BlockSpec((1, f), lambda i, p, h: (0, 0)),
            pl.BlockSpec((1, f), lambda i, p, h: (0, 0)),
            pl.BlockSpec((1, f), lambda i, p, h: (0, 0)),
            pl.BlockSpec((1, f, n, n), lambda i, p, h: (i, 0, 0, 0)),
        ],
        out_specs=pl.BlockSpec(
            (1, fh, n, n),
            lambda i, p, h: (i, jnp.where(p == 0, 0, h), 0, 0)),
        out_shape=jax.ShapeDtypeStruct((b, f, n, n), jnp.float32),
        scratch_shapes=[pltpu.VMEM((2, n), jnp.float32)],
        compiler_params=pltpu.CompilerParams(
            dimension_semantics=("parallel", "arbitrary", "arbitrary")),
    )(mask3, w2, wexp2, wb2, bias2, x)
    return out


# speedup vs baseline: 1.3185x; 1.3185x over previous
import jax
import jax.numpy as jnp
from jax.experimental import pallas as pl
from jax.experimental.pallas import tpu as pltpu

EPS = 1e-09
C = 8


def _fused_kernel(mask_ref, w_ref, wexp_ref, wbias_ref, bias_ref, x_ref,
                  o_ref, xbuf, obuf, insem, outsem):
    j = pl.program_id(1)
    nsteps = pl.num_programs(1)
    b = pl.program_id(0) * nsteps + j
    nf = x_ref.shape[1]
    n = x_ref.shape[2]
    cur = jax.lax.rem(j, 2)
    nxt = jax.lax.rem(j + 1, 2)

    def dma_in(slot, bb):
        pltpu.make_async_copy(x_ref.at[bb], xbuf.at[slot],
                              insem.at[slot]).start()

    def wait_in(slot):
        pltpu.make_async_copy(x_ref.at[0], xbuf.at[slot],
                              insem.at[slot]).wait()

    def dma_out(slot, bb, c):
        pltpu.make_async_copy(obuf.at[slot],
                              o_ref.at[bb, pl.ds(c * C, C)],
                              outsem.at[slot]).start()

    def wait_out(slot):
        pltpu.make_async_copy(obuf.at[slot],
                              o_ref.at[0, pl.ds(0, C)],
                              outsem.at[slot]).wait()

    @pl.when(j == 0)
    def _():
        dma_in(cur, b)

    @pl.when(j + 1 < nsteps)
    def _():
        dma_in(nxt, b + 1)

    wait_in(cur)

    m = mask_ref[0]
    md = m * m
    lane = jax.lax.broadcasted_iota(jnp.int32, (1, n), 1)
    ii = jax.lax.broadcasted_iota(jnp.int32, (n, n), 0)
    kk = jax.lax.broadcasted_iota(jnp.int32, (n, n), 1)
    eye = ii == kk

    e2 = jnp.where(eye, md, 0.0)
    trvec = jnp.zeros((1, n), jnp.float32)
    qs = jnp.zeros((n, n), jnp.float32)
    for fidx in range(nf):
        a = xbuf[cur, fidx]
        trvec = trvec + jnp.where(lane == fidx, jnp.sum(a * e2), 0.0)
        qs = qs + a * jnp.transpose(a)

    cnt = jnp.sum(md)
    cnt2 = jnp.maximum(cnt - 1.0, 1.0)
    s1 = jnp.sum(trvec)
    s3 = jnp.sum(trvec * trvec)
    s2 = jnp.sum(qs * md)
    mean_b = s1 / (cnt * nf)
    var_b = (s2 - s3 / cnt) / (cnt2 * nf)
    inv = jax.lax.rsqrt(var_b + EPS)

    scale_vec = w_ref[...] * jnp.exp(wexp_ref[...]) + wbias_ref[...]
    flane = jax.lax.broadcasted_iota(jnp.int32, (1, nf), 1)
    e1 = jnp.where(eye, 1.0, 0.0)
    m2i = (jnp.transpose(m) * m) * inv
    me1 = mean_b * e1

    for c in range(nf // C):
        oslot = c % 2
        if c >= 2:
            wait_out(oslot)
        else:
            @pl.when(j > 0)
            def _():
                wait_out(oslot)
        for g in range(C):
            fidx = c * C + g
            fsel = flane == fidx
            s = jnp.sum(jnp.where(fsel, scale_vec, 0.0))
            bf = jnp.sum(jnp.where(fsel, bias_ref[...], 0.0))
            a = xbuf[cur, fidx]
            obuf[oslot, g] = ((a - me1) * m2i) * s + bf * e1
        dma_out(oslot, b, c)

    @pl.when(j == nsteps - 1)
    def _():
        wait_out(0)
        wait_out(1)


def kernel(x, mask, weight, weight_exp, weight_bias, bias):
    b, f, n, _ = x.shape
    half = b // 2
    mask3 = mask.reshape(b, 1, n)
    w2 = weight.reshape(1, f)
    wexp2 = weight_exp.reshape(1, f)
    wb2 = weight_bias.reshape(1, f)
    bias2 = bias.reshape(1, f)

    out = pl.pallas_call(
        _fused_kernel,
        grid=(2, half),
        in_specs=[
            pl.BlockSpec((1, 1, n), lambda i, j: (i * half + j, 0, 0)),
            pl.BlockSpec((1, f), lambda i, j: (0, 0)),
            pl.BlockSpec((1, f), lambda i, j: (0, 0)),
            pl.BlockSpec((1, f), lambda i, j: (0, 0)),
            pl.BlockSpec((1, f), lambda i, j: (0, 0)),
            pl.BlockSpec(memory_space=pl.ANY),
        ],
        out_specs=pl.BlockSpec(memory_space=pl.ANY),
        out_shape=jax.ShapeDtypeStruct((b, f, n, n), jnp.float32),
        scratch_shapes=[
            pltpu.VMEM((2, f, n, n), jnp.float32),
            pltpu.VMEM((2, C, n, n), jnp.float32),
            pltpu.SemaphoreType.DMA((2,)),
            pltpu.SemaphoreType.DMA((2,)),
        ],
        compiler_params=pltpu.CompilerParams(
            dimension_semantics=("parallel", "arbitrary")),
    )(mask3, w2, wexp2, wb2, bias2, x)
    return out


# speedup vs baseline: 1.3344x; 1.0121x over previous
import jax
import jax.numpy as jnp
from jax.experimental import pallas as pl
from jax.experimental.pallas import tpu as pltpu

EPS = 1e-09
C = 8


def _fused_kernel(mask_ref, w_ref, wexp_ref, wbias_ref, bias_ref, x_ref,
                  o_ref, xbuf, obuf, insem, outsem):
    j = pl.program_id(1)
    nsteps = pl.num_programs(1)
    b = pl.program_id(0) * nsteps + j
    nf = x_ref.shape[1]
    n = x_ref.shape[2]
    nc = nf // C
    cur = jax.lax.rem(j, 2)
    nxt = jax.lax.rem(j + 1, 2)

    def dma_in(slot, bb, c):
        pltpu.make_async_copy(x_ref.at[bb, pl.ds(c * C, C)],
                              xbuf.at[slot, pl.ds(c * C, C)],
                              insem.at[slot, c]).start()

    def wait_in(slot, c):
        pltpu.make_async_copy(x_ref.at[0, pl.ds(0, C)],
                              xbuf.at[slot, pl.ds(0, C)],
                              insem.at[slot, c]).wait()

    def dma_out(slot, bb, c):
        pltpu.make_async_copy(obuf.at[slot],
                              o_ref.at[bb, pl.ds(c * C, C)],
                              outsem.at[slot]).start()

    def wait_out(slot):
        pltpu.make_async_copy(obuf.at[slot],
                              o_ref.at[0, pl.ds(0, C)],
                              outsem.at[slot]).wait()

    @pl.when(j == 0)
    def _():
        for c in range(nc):
            dma_in(cur, b, c)

    m = mask_ref[0]
    md = m * m
    lane = jax.lax.broadcasted_iota(jnp.int32, (1, n), 1)
    ii = jax.lax.broadcasted_iota(jnp.int32, (n, n), 0)
    kk = jax.lax.broadcasted_iota(jnp.int32, (n, n), 1)
    eye = ii == kk

    e2 = jnp.where(eye, md, 0.0)
    trvec = jnp.zeros((1, n), jnp.float32)
    qs = jnp.zeros((n, n), jnp.float32)
    for c in range(nc):
        wait_in(cur, c)

        @pl.when(j + 1 < nsteps)
        def _():
            dma_in(nxt, b + 1, c)

        for g in range(C):
            fidx = c * C + g
            a = xbuf[cur, fidx]
            trvec = trvec + jnp.where(lane == fidx, jnp.sum(a * e2), 0.0)
            qs = qs + a * jnp.transpose(a)

    cnt = jnp.sum(md)
    cnt2 = jnp.maximum(cnt - 1.0, 1.0)
    s1 = jnp.sum(trvec)
    s3 = jnp.sum(trvec * trvec)
    s2 = jnp.sum(qs * md)
    mean_b = s1 / (cnt * nf)
    var_b = (s2 - s3 / cnt) / (cnt2 * nf)
    inv = jax.lax.rsqrt(var_b + EPS)

    scale_vec = w_ref[...] * jnp.exp(wexp_ref[...]) + wbias_ref[...]
    flane = jax.lax.broadcasted_iota(jnp.int32, (1, nf), 1)
    e1 = jnp.where(eye, 1.0, 0.0)
    m2i = (jnp.transpose(m) * m) * inv
    me1 = mean_b * e1

    for c in range(nc):
        oslot = c % 2
        if c >= 2:
            wait_out(oslot)
        else:
            @pl.when(j > 0)
            def _():
                wait_out(oslot)
        for g in range(C):
            fidx = c * C + g
            fsel = flane == fidx
            s = jnp.sum(jnp.where(fsel, scale_vec, 0.0))
            bf = jnp.sum(jnp.where(fsel, bias_ref[...], 0.0))
            a = xbuf[cur, fidx]
            obuf[oslot, g] = ((a - me1) * m2i) * s + bf * e1
        dma_out(oslot, b, c)

    @pl.when(j == nsteps - 1)
    def _():
        wait_out(0)
        wait_out(1)


def kernel(x, mask, weight, weight_exp, weight_bias, bias):
    b, f, n, _ = x.shape
    half = b // 2
    mask3 = mask.reshape(b, 1, n)
    w2 = weight.reshape(1, f)
    wexp2 = weight_exp.reshape(1, f)
    wb2 = weight_bias.reshape(1, f)
    bias2 = bias.reshape(1, f)

    out = pl.pallas_call(
        _fused_kernel,
        grid=(2, half),
        in_specs=[
            pl.BlockSpec((1, 1, n), lambda i, j: (i * half + j, 0, 0)),
            pl.BlockSpec((1, f), lambda i, j: (0, 0)),
            pl.BlockSpec((1, f), lambda i, j: (0, 0)),
            pl.BlockSpec((1, f), lambda i, j: (0, 0)),
            pl.BlockSpec((1, f), lambda i, j: (0, 0)),
            pl.BlockSpec(memory_space=pl.ANY),
        ],
        out_specs=pl.BlockSpec(memory_space=pl.ANY),
        out_shape=jax.ShapeDtypeStruct((b, f, n, n), jnp.float32),
        scratch_shapes=[
            pltpu.VMEM((2, f, n, n), jnp.float32),
            pltpu.VMEM((2, C, n, n), jnp.float32),
            pltpu.SemaphoreType.DMA((2, f // C)),
            pltpu.SemaphoreType.DMA((2,)),
        ],
        compiler_params=pltpu.CompilerParams(
            dimension_semantics=("parallel", "arbitrary")),
    )(mask3, w2, wexp2, wb2, bias2, x)
    return out


# speedup vs baseline: 1.5517x; 1.1628x over previous
import jax
import jax.numpy as jnp
from jax.experimental import pallas as pl
from jax.experimental.pallas import tpu as pltpu

EPS = 1e-09
C = 8


def _fused_kernel(mask_ref, w_ref, wexp_ref, wbias_ref, bias_ref, x_ref,
                  o_ref, xbuf, obuf, insem, outsem):
    j = pl.program_id(1)
    nsteps = pl.num_programs(1)
    b = pl.program_id(0) * nsteps + j
    nf = x_ref.shape[1]
    n = x_ref.shape[2]
    nc = nf // C
    cur = jax.lax.rem(j, 2)
    nxt = jax.lax.rem(j + 1, 2)

    def dma_in(slot, bb, c):
        pltpu.make_async_copy(x_ref.at[bb, pl.ds(c * C, C)],
                              xbuf.at[slot, pl.ds(c * C, C)],
                              insem.at[slot, c]).start()

    def wait_in(slot, c):
        pltpu.make_async_copy(x_ref.at[0, pl.ds(0, C)],
                              xbuf.at[slot, pl.ds(0, C)],
                              insem.at[slot, c]).wait()

    def dma_out(slot, bb, c):
        pltpu.make_async_copy(obuf.at[slot],
                              o_ref.at[bb, pl.ds(c * C, C)],
                              outsem.at[slot]).start()

    def wait_out(slot):
        pltpu.make_async_copy(obuf.at[slot],
                              o_ref.at[0, pl.ds(0, C)],
                              outsem.at[slot]).wait()

    @pl.when(j == 0)
    def _():
        for c in range(nc):
            dma_in(cur, b, c)

    @pl.when(j + 1 < nsteps)
    def _():
        for c in range(nc):
            dma_in(nxt, b + 1, c)

    m = mask_ref[0]
    md = m * m
    lane = jax.lax.broadcasted_iota(jnp.int32, (1, n), 1)
    ii = jax.lax.broadcasted_iota(jnp.int32, (n, n), 0)
    kk = jax.lax.broadcasted_iota(jnp.int32, (n, n), 1)
    eye = ii == kk

    e2 = jnp.where(eye, md, 0.0)
    trvec = jnp.zeros((1, n), jnp.float32)
    qs = jnp.zeros((n, n), jnp.float32)
    for c in range(nc):
        wait_in(cur, c)
        for g in range(C):
            fidx = c * C + g
            a = xbuf[cur, fidx]
            trvec = trvec + jnp.where(lane == fidx, jnp.sum(a * e2), 0.0)
            qs = qs + a * jnp.transpose(a)

    cnt = jnp.sum(md)
    cnt2 = jnp.maximum(cnt - 1.0, 1.0)
    s1 = jnp.sum(trvec)
    s3 = jnp.sum(trvec * trvec)
    s2 = jnp.sum(qs * md)
    mean_b = s1 / (cnt * nf)
    var_b = (s2 - s3 / cnt) / (cnt2 * nf)
    inv = jax.lax.rsqrt(var_b + EPS)

    scale_vec = w_ref[...] * jnp.exp(wexp_ref[...]) + wbias_ref[...]
    flane = jax.lax.broadcasted_iota(jnp.int32, (1, nf), 1)
    e1 = jnp.where(eye, 1.0, 0.0)
    m2i = (jnp.transpose(m) * m) * inv
    me1 = mean_b * e1

    for c in range(nc):
        @pl.when(j > 0)
        def _():
            wait_out(c)
        for g in range(C):
            fidx = c * C + g
            fsel = flane == fidx
            s = jnp.sum(jnp.where(fsel, scale_vec, 0.0))
            bf = jnp.sum(jnp.where(fsel, bias_ref[...], 0.0))
            a = xbuf[cur, fidx]
            obuf[c, g] = ((a - me1) * m2i) * s + bf * e1
        dma_out(c, b, c)

    @pl.when(j == nsteps - 1)
    def _():
        for c in range(nc):
            wait_out(c)


def kernel(x, mask, weight, weight_exp, weight_bias, bias):
    b, f, n, _ = x.shape
    half = b // 2
    mask3 = mask.reshape(b, 1, n)
    w2 = weight.reshape(1, f)
    wexp2 = weight_exp.reshape(1, f)
    wb2 = weight_bias.reshape(1, f)
    bias2 = bias.reshape(1, f)

    out = pl.pallas_call(
        _fused_kernel,
        grid=(2, half),
        in_specs=[
            pl.BlockSpec((1, 1, n), lambda i, j: (i * half + j, 0, 0)),
            pl.BlockSpec((1, f), lambda i, j: (0, 0)),
            pl.BlockSpec((1, f), lambda i, j: (0, 0)),
            pl.BlockSpec((1, f), lambda i, j: (0, 0)),
            pl.BlockSpec((1, f), lambda i, j: (0, 0)),
            pl.BlockSpec(memory_space=pl.ANY),
        ],
        out_specs=pl.BlockSpec(memory_space=pl.ANY),
        out_shape=jax.ShapeDtypeStruct((b, f, n, n), jnp.float32),
        scratch_shapes=[
            pltpu.VMEM((2, f, n, n), jnp.float32),
            pltpu.VMEM((f // C, C, n, n), jnp.float32),
            pltpu.SemaphoreType.DMA((2, f // C)),
            pltpu.SemaphoreType.DMA((f // C,)),
        ],
        compiler_params=pltpu.CompilerParams(
            dimension_semantics=("parallel", "arbitrary")),
    )(mask3, w2, wexp2, wb2, bias2, x)
    return out


# speedup vs baseline: 1.5845x; 1.0211x over previous
import jax
import jax.numpy as jnp
from jax.experimental import pallas as pl
from jax.experimental.pallas import tpu as pltpu

EPS = 1e-09
C = 16


def _fused_kernel(mask_ref, w_ref, wexp_ref, wbias_ref, bias_ref, x_ref,
                  o_ref, xbuf, obuf, insem, outsem):
    j = pl.program_id(1)
    nsteps = pl.num_programs(1)
    b = pl.program_id(0) * nsteps + j
    nf = x_ref.shape[1]
    n = x_ref.shape[2]
    nc = nf // C
    cur = jax.lax.rem(j, 2)
    nxt = jax.lax.rem(j + 1, 2)

    def dma_in(slot, bb, c):
        pltpu.make_async_copy(x_ref.at[bb, pl.ds(c * C, C)],
                              xbuf.at[slot, pl.ds(c * C, C)],
                              insem.at[slot, c]).start()

    def wait_in(slot, c):
        pltpu.make_async_copy(x_ref.at[0, pl.ds(0, C)],
                              xbuf.at[slot, pl.ds(0, C)],
                              insem.at[slot, c]).wait()

    def dma_out(slot, bb, c):
        pltpu.make_async_copy(obuf.at[slot],
                              o_ref.at[bb, pl.ds(c * C, C)],
                              outsem.at[slot]).start()

    def wait_out(slot):
        pltpu.make_async_copy(obuf.at[slot],
                              o_ref.at[0, pl.ds(0, C)],
                              outsem.at[slot]).wait()

    @pl.when(j == 0)
    def _():
        for c in range(nc):
            dma_in(cur, b, c)

    @pl.when(j + 1 < nsteps)
    def _():
        for c in range(nc):
            dma_in(nxt, b + 1, c)

    m = mask_ref[0]
    md = m * m
    lane = jax.lax.broadcasted_iota(jnp.int32, (1, n), 1)
    ii = jax.lax.broadcasted_iota(jnp.int32, (n, n), 0)
    kk = jax.lax.broadcasted_iota(jnp.int32, (n, n), 1)
    eye = ii == kk

    e2 = jnp.where(eye, md, 0.0)
    trvec = jnp.zeros((1, n), jnp.float32)
    qs = jnp.zeros((n, n), jnp.float32)
    for c in range(nc):
        wait_in(cur, c)
        for g in range(C):
            fidx = c * C + g
            a = xbuf[cur, fidx]
            trvec = trvec + jnp.where(lane == fidx, jnp.sum(a * e2), 0.0)
            qs = qs + a * jnp.transpose(a)

    cnt = jnp.sum(md)
    cnt2 = jnp.maximum(cnt - 1.0, 1.0)
    s1 = jnp.sum(trvec)
    s3 = jnp.sum(trvec * trvec)
    s2 = jnp.sum(qs * md)
    mean_b = s1 / (cnt * nf)
    var_b = (s2 - s3 / cnt) / (cnt2 * nf)
    inv = jax.lax.rsqrt(var_b + EPS)

    scale_vec = w_ref[...] * jnp.exp(wexp_ref[...]) + wbias_ref[...]
    flane = jax.lax.broadcasted_iota(jnp.int32, (1, nf), 1)
    e1 = jnp.where(eye, 1.0, 0.0)
    m2i = (jnp.transpose(m) * m) * inv
    me1 = mean_b * e1

    for c in range(nc):
        @pl.when(j > 0)
        def _():
            wait_out(c)
        for g in range(C):
            fidx = c * C + g
            fsel = flane == fidx
            s = jnp.sum(jnp.where(fsel, scale_vec, 0.0))
            bf = jnp.sum(jnp.where(fsel, bias_ref[...], 0.0))
            a = xbuf[cur, fidx]
            obuf[c, g] = ((a - me1) * m2i) * s + bf * e1
        dma_out(c, b, c)

    @pl.when(j == nsteps - 1)
    def _():
        for c in range(nc):
            wait_out(c)


def kernel(x, mask, weight, weight_exp, weight_bias, bias):
    b, f, n, _ = x.shape
    half = b // 2
    mask3 = mask.reshape(b, 1, n)
    w2 = weight.reshape(1, f)
    wexp2 = weight_exp.reshape(1, f)
    wb2 = weight_bias.reshape(1, f)
    bias2 = bias.reshape(1, f)

    out = pl.pallas_call(
        _fused_kernel,
        grid=(2, half),
        in_specs=[
            pl.BlockSpec((1, 1, n), lambda i, j: (i * half + j, 0, 0)),
            pl.BlockSpec((1, f), lambda i, j: (0, 0)),
            pl.BlockSpec((1, f), lambda i, j: (0, 0)),
            pl.BlockSpec((1, f), lambda i, j: (0, 0)),
            pl.BlockSpec((1, f), lambda i, j: (0, 0)),
            pl.BlockSpec(memory_space=pl.ANY),
        ],
        out_specs=pl.BlockSpec(memory_space=pl.ANY),
        out_shape=jax.ShapeDtypeStruct((b, f, n, n), jnp.float32),
        scratch_shapes=[
            pltpu.VMEM((2, f, n, n), jnp.float32),
            pltpu.VMEM((f // C, C, n, n), jnp.float32),
            pltpu.SemaphoreType.DMA((2, f // C)),
            pltpu.SemaphoreType.DMA((f // C,)),
        ],
        compiler_params=pltpu.CompilerParams(
            dimension_semantics=("parallel", "arbitrary")),
    )(mask3, w2, wexp2, wb2, bias2, x)
    return out


# speedup vs baseline: 1.6303x; 1.0289x over previous
import jax
import jax.numpy as jnp
from jax.experimental import pallas as pl
from jax.experimental.pallas import tpu as pltpu

EPS = 1e-09
C = 16


def _fused_kernel(mask_ref, w_ref, wexp_ref, wbias_ref, bias_ref, x_ref,
                  o_ref, xbuf, obuf, insem, outsem):
    j = pl.program_id(1)
    nsteps = pl.num_programs(1)
    b = pl.program_id(0) * nsteps + j
    nf = x_ref.shape[1]
    n = x_ref.shape[2]
    nc = nf // C
    cur = jax.lax.rem(j, 2)
    nxt = jax.lax.rem(j + 1, 2)

    def dma_in(slot, bb, c):
        pltpu.make_async_copy(x_ref.at[bb, pl.ds(c * C, C)],
                              xbuf.at[slot, pl.ds(c * C, C)],
                              insem.at[slot, c]).start()

    def wait_in(slot, c):
        pltpu.make_async_copy(x_ref.at[0, pl.ds(0, C)],
                              xbuf.at[slot, pl.ds(0, C)],
                              insem.at[slot, c]).wait()

    def dma_out(slot, bb, c):
        pltpu.make_async_copy(obuf.at[slot],
                              o_ref.at[bb, pl.ds(c * C, C)],
                              outsem.at[slot]).start()

    def wait_out(slot):
        pltpu.make_async_copy(obuf.at[slot],
                              o_ref.at[0, pl.ds(0, C)],
                              outsem.at[slot]).wait()

    @pl.when(j == 0)
    def _():
        for c in range(nc):
            dma_in(cur, b, c)

    @pl.when(j + 1 < nsteps)
    def _():
        for c in range(nc):
            dma_in(nxt, b + 1, c)

    m = mask_ref[0]
    md = m * m
    lane = jax.lax.broadcasted_iota(jnp.int32, (1, n), 1)
    ii = jax.lax.broadcasted_iota(jnp.int32, (n, n), 0)
    kk = jax.lax.broadcasted_iota(jnp.int32, (n, n), 1)
    eye = ii == kk

    e2 = jnp.where(eye, md, 0.0)
    trvec = jnp.zeros((1, n), jnp.float32)
    s2 = jnp.float32(0.0)
    for c in range(nc):
        wait_in(cur, c)
        for g in range(C):
            fidx = c * C + g
            a = xbuf[cur, fidx]
            trvec = trvec + jnp.where(lane == fidx, jnp.sum(a * e2), 0.0)
            s2 = s2 + jnp.sum(a * jnp.transpose(a) * md)

    cnt = jnp.sum(md)
    cnt2 = jnp.maximum(cnt - 1.0, 1.0)
    s1 = jnp.sum(trvec)
    s3 = jnp.sum(trvec * trvec)
    mean_b = s1 / (cnt * nf)
    var_b = (s2 - s3 / cnt) / (cnt2 * nf)
    inv = jax.lax.rsqrt(var_b + EPS)

    scale_vec = w_ref[...] * jnp.exp(wexp_ref[...]) + wbias_ref[...]
    flane = jax.lax.broadcasted_iota(jnp.int32, (1, nf), 1)
    e1 = jnp.where(eye, 1.0, 0.0)
    m2i = (jnp.transpose(m) * m) * inv
    me1 = mean_b * e1

    for c in range(nc):
        @pl.when(j > 0)
        def _():
            wait_out(c)
        for g in range(C):
            fidx = c * C + g
            fsel = flane == fidx
            s = jnp.sum(jnp.where(fsel, scale_vec, 0.0))
            bf = jnp.sum(jnp.where(fsel, bias_ref[...], 0.0))
            a = xbuf[cur, fidx]
            obuf[c, g] = ((a - me1) * m2i) * s + bf * e1
        dma_out(c, b, c)

    @pl.when(j == nsteps - 1)
    def _():
        for c in range(nc):
            wait_out(c)


def kernel(x, mask, weight, weight_exp, weight_bias, bias):
    b, f, n, _ = x.shape
    half = b // 2
    mask3 = mask.reshape(b, 1, n)
    w2 = weight.reshape(1, f)
    wexp2 = weight_exp.reshape(1, f)
    wb2 = weight_bias.reshape(1, f)
    bias2 = bias.reshape(1, f)

    out = pl.pallas_call(
        _fused_kernel,
        grid=(2, half),
        in_specs=[
            pl.BlockSpec((1, 1, n), lambda i, j: (i * half + j, 0, 0)),
            pl.BlockSpec((1, f), lambda i, j: (0, 0)),
            pl.BlockSpec((1, f), lambda i, j: (0, 0)),
            pl.BlockSpec((1, f), lambda i, j: (0, 0)),
            pl.BlockSpec((1, f), lambda i, j: (0, 0)),
            pl.BlockSpec(memory_space=pl.ANY),
        ],
        out_specs=pl.BlockSpec(memory_space=pl.ANY),
        out_shape=jax.ShapeDtypeStruct((b, f, n, n), jnp.float32),
        scratch_shapes=[
            pltpu.VMEM((2, f, n, n), jnp.float32),
            pltpu.VMEM((f // C, C, n, n), jnp.float32),
            pltpu.SemaphoreType.DMA((2, f // C)),
            pltpu.SemaphoreType.DMA((f // C,)),
        ],
        compiler_params=pltpu.CompilerParams(
            dimension_semantics=("parallel", "arbitrary")),
    )(mask3, w2, wexp2, wb2, bias2, x)
    return out


# speedup vs baseline: 1.7499x; 1.0734x over previous
import jax
import jax.numpy as jnp
from jax.experimental import pallas as pl
from jax.experimental.pallas import tpu as pltpu

EPS = 1e-09
C = 16


def _fused_kernel(mask_ref, w_ref, wexp_ref, wbias_ref, bias_ref, x_ref,
                  o_ref, xbuf, obuf, insem, outsem):
    j = pl.program_id(1)
    nsteps = pl.num_programs(1)
    b = pl.program_id(0) * nsteps + j
    nf = x_ref.shape[1]
    n = x_ref.shape[2]
    nc = nf // C
    cur = jax.lax.rem(j, 2)
    nxt = jax.lax.rem(j + 1, 2)

    def dma_in(slot, bb, c):
        pltpu.make_async_copy(x_ref.at[bb, pl.ds(c * C, C)],
                              xbuf.at[slot, pl.ds(c * C, C)],
                              insem.at[slot, c]).start()

    def wait_in(slot, c):
        pltpu.make_async_copy(x_ref.at[0, pl.ds(0, C)],
                              xbuf.at[slot, pl.ds(0, C)],
                              insem.at[slot, c]).wait()

    def dma_out(slot, bb, c):
        pltpu.make_async_copy(obuf.at[slot],
                              o_ref.at[bb, pl.ds(c * C, C)],
                              outsem.at[slot]).start()

    def wait_out(slot):
        pltpu.make_async_copy(obuf.at[slot],
                              o_ref.at[0, pl.ds(0, C)],
                              outsem.at[slot]).wait()

    @pl.when(j == 0)
    def _():
        for c in range(nc):
            dma_in(cur, b, c)

    @pl.when(j + 1 < nsteps)
    def _():
        for c in range(nc):
            dma_in(nxt, b + 1, c)

    m = mask_ref[0]
    md = m * m
    lane = jax.lax.broadcasted_iota(jnp.int32, (1, n), 1)
    ii = jax.lax.broadcasted_iota(jnp.int32, (n, n), 0)
    kk = jax.lax.broadcasted_iota(jnp.int32, (n, n), 1)
    eye = ii == kk

    e2 = jnp.where(eye, md, 0.0)
    trvec = jnp.zeros((1, n), jnp.float32)
    s2 = jnp.float32(0.0)
    for c in range(nc):
        wait_in(cur, c)
        for g in range(C):
            fidx = c * C + g
            a = xbuf[cur, fidx]
            trvec = trvec + jnp.where(lane == fidx, jnp.sum(a * e2), 0.0)
            s2 = s2 + jnp.sum(a * jnp.transpose(a) * md)

    cnt = jnp.sum(md)
    cnt2 = jnp.maximum(cnt - 1.0, 1.0)
    s1 = jnp.sum(trvec)
    s3 = jnp.sum(trvec * trvec)
    mean_b = s1 / (cnt * nf)
    var_b = (s2 - s3 / cnt) / (cnt2 * nf)
    inv = jax.lax.rsqrt(var_b + EPS)

    scale_vec = w_ref[...] * jnp.exp(wexp_ref[...]) + wbias_ref[...]
    flane = jax.lax.broadcasted_iota(jnp.int32, (1, nf), 1)
    e1 = jnp.where(eye, 1.0, 0.0)
    m2i = (jnp.transpose(m) * m) * inv
    me1 = mean_b * e1

    hn = n // 2
    for c in range(nc):
        @pl.when(j > 0)
        def _():
            wait_out(c)
        svals = []
        bvals = []
        for g in range(C):
            fidx = c * C + g
            fsel = flane == fidx
            svals.append(jnp.sum(jnp.where(fsel, scale_vec, 0.0)))
            bvals.append(jnp.sum(jnp.where(fsel, bias_ref[...], 0.0)))
        for rb in range(2):
            rs = slice(rb * hn, (rb + 1) * hn)
            me1_h = me1[rs, :]
            m2i_h = m2i[rs, :]
            e1_h = e1[rs, :]
            for g in range(C):
                fidx = c * C + g
                a_h = xbuf[cur, fidx, rs, :]
                obuf[c, g, rs, :] = (((a_h - me1_h) * m2i_h) * svals[g]
                                     + bvals[g] * e1_h)
        dma_out(c, b, c)

    @pl.when(j == nsteps - 1)
    def _():
        for c in range(nc):
            wait_out(c)


def kernel(x, mask, weight, weight_exp, weight_bias, bias):
    b, f, n, _ = x.shape
    half = b // 2
    mask3 = mask.reshape(b, 1, n)
    w2 = weight.reshape(1, f)
    wexp2 = weight_exp.reshape(1, f)
    wb2 = weight_bias.reshape(1, f)
    bias2 = bias.reshape(1, f)

    out = pl.pallas_call(
        _fused_kernel,
        grid=(2, half),
        in_specs=[
            pl.BlockSpec((1, 1, n), lambda i, j: (i * half + j, 0, 0)),
            pl.BlockSpec((1, f), lambda i, j: (0, 0)),
            pl.BlockSpec((1, f), lambda i, j: (0, 0)),
            pl.BlockSpec((1, f), lambda i, j: (0, 0)),
            pl.BlockSpec((1, f), lambda i, j: (0, 0)),
            pl.BlockSpec(memory_space=pl.ANY),
        ],
        out_specs=pl.BlockSpec(memory_space=pl.ANY),
        out_shape=jax.ShapeDtypeStruct((b, f, n, n), jnp.float32),
        scratch_shapes=[
            pltpu.VMEM((2, f, n, n), jnp.float32),
            pltpu.VMEM((f // C, C, n, n), jnp.float32),
            pltpu.SemaphoreType.DMA((2, f // C)),
            pltpu.SemaphoreType.DMA((f // C,)),
        ],
        compiler_params=pltpu.CompilerParams(
            dimension_semantics=("parallel", "arbitrary")),
    )(mask3, w2, wexp2, wb2, bias2, x)
    return out


# speedup vs baseline: 1.8925x; 1.0815x over previous
import jax
import jax.numpy as jnp
from jax.experimental import pallas as pl
from jax.experimental.pallas import tpu as pltpu

EPS = 1e-09
C = 16


def _fused_kernel(mask_ref, w_ref, wexp_ref, wbias_ref, bias_ref, x_ref,
                  o_ref, xbuf, obuf, insem, outsem):
    j = pl.program_id(1)
    nsteps = pl.num_programs(1)
    b = pl.program_id(0) * nsteps + j
    nf = x_ref.shape[1]
    n = x_ref.shape[2]
    nc = nf // C
    cur = jax.lax.rem(j, 2)
    nxt = jax.lax.rem(j + 1, 2)

    def dma_in(slot, bb, c):
        pltpu.make_async_copy(x_ref.at[bb, pl.ds(c * C, C)],
                              xbuf.at[slot, pl.ds(c * C, C)],
                              insem.at[slot, c]).start()

    def wait_in(slot, c):
        pltpu.make_async_copy(x_ref.at[0, pl.ds(0, C)],
                              xbuf.at[slot, pl.ds(0, C)],
                              insem.at[slot, c]).wait()

    def dma_out(slot, bb, c):
        pltpu.make_async_copy(obuf.at[slot],
                              o_ref.at[bb, pl.ds(c * C, C)],
                              outsem.at[slot]).start()

    def wait_out(slot):
        pltpu.make_async_copy(obuf.at[slot],
                              o_ref.at[0, pl.ds(0, C)],
                              outsem.at[slot]).wait()

    @pl.when(j == 0)
    def _():
        for c in range(nc):
            dma_in(cur, b, c)

    @pl.when(j + 1 < nsteps)
    def _():
        for c in range(nc):
            dma_in(nxt, b + 1, c)

    m = mask_ref[0]
    md = m * m
    lane = jax.lax.broadcasted_iota(jnp.int32, (1, n), 1)
    ii = jax.lax.broadcasted_iota(jnp.int32, (n, n), 0)
    kk = jax.lax.broadcasted_iota(jnp.int32, (n, n), 1)
    eye = ii == kk

    hn = n // 2
    lo = slice(0, hn)
    hi = slice(hn, n)
    e2 = jnp.where(eye, md, 0.0)
    e2a = e2[lo, lo]
    e2b = e2[hi, hi]
    trvec = jnp.zeros((1, n), jnp.float32)
    s2 = jnp.float32(0.0)
    for c in range(nc):
        wait_in(cur, c)
        for g in range(C):
            fidx = c * C + g
            a = xbuf[cur, fidx]
            tr_g = jnp.sum(a[lo, lo] * e2a) + jnp.sum(a[hi, hi] * e2b)
            trvec = trvec + jnp.where(lane == fidx, tr_g, 0.0)
            s2 = s2 + jnp.sum(a * jnp.transpose(a) * md)

    cnt = jnp.sum(md)
    cnt2 = jnp.maximum(cnt - 1.0, 1.0)
    s1 = jnp.sum(trvec)
    s3 = jnp.sum(trvec * trvec)
    mean_b = s1 / (cnt * nf)
    var_b = (s2 - s3 / cnt) / (cnt2 * nf)
    inv = jax.lax.rsqrt(var_b + EPS)

    scale_vec = w_ref[...] * jnp.exp(wexp_ref[...]) + wbias_ref[...]
    flane = jax.lax.broadcasted_iota(jnp.int32, (1, nf), 1)
    e1 = jnp.where(eye, 1.0, 0.0)
    m2i = (jnp.transpose(m) * m) * inv
    me1 = mean_b * e1

    for c in range(nc):
        @pl.when(j > 0)
        def _():
            wait_out(c)
        svals = []
        bvals = []
        for g in range(C):
            fidx = c * C + g
            fsel = flane == fidx
            svals.append(jnp.sum(jnp.where(fsel, scale_vec, 0.0)))
            bvals.append(jnp.sum(jnp.where(fsel, bias_ref[...], 0.0)))
        for rb in range(2):
            rs = lo if rb == 0 else hi
            for cb in range(2):
                cs = lo if cb == 0 else hi
                m2i_q = m2i[rs, cs]
                if cb == rb:
                    me1_q = me1[rs, cs]
                    e1_q = e1[rs, cs]
                    for g in range(C):
                        fidx = c * C + g
                        a_q = xbuf[cur, fidx, rs, cs]
                        obuf[c, g, rs, cs] = (
                            ((a_q - me1_q) * m2i_q) * svals[g]
                            + bvals[g] * e1_q)
                else:
                    for g in range(C):
                        fidx = c * C + g
                        a_q = xbuf[cur, fidx, rs, cs]
                        obuf[c, g, rs, cs] = (a_q * m2i_q) * svals[g]
        dma_out(c, b, c)

    @pl.when(j == nsteps - 1)
    def _():
        for c in range(nc):
            wait_out(c)


def kernel(x, mask, weight, weight_exp, weight_bias, bias):
    b, f, n, _ = x.shape
    half = b // 2
    mask3 = mask.reshape(b, 1, n)
    w2 = weight.reshape(1, f)
    wexp2 = weight_exp.reshape(1, f)
    wb2 = weight_bias.reshape(1, f)
    bias2 = bias.reshape(1, f)

    out = pl.pallas_call(
        _fused_kernel,
        grid=(2, half),
        in_specs=[
            pl.BlockSpec((1, 1, n), lambda i, j: (i * half + j, 0, 0)),
            pl.BlockSpec((1, f), lambda i, j: (0, 0)),
            pl.BlockSpec((1, f), lambda i, j: (0, 0)),
            pl.BlockSpec((1, f), lambda i, j: (0, 0)),
            pl.BlockSpec((1, f), lambda i, j: (0, 0)),
            pl.BlockSpec(memory_space=pl.ANY),
        ],
        out_specs=pl.BlockSpec(memory_space=pl.ANY),
        out_shape=jax.ShapeDtypeStruct((b, f, n, n), jnp.float32),
        scratch_shapes=[
            pltpu.VMEM((2, f, n, n), jnp.float32),
            pltpu.VMEM((f // C, C, n, n), jnp.float32),
            pltpu.SemaphoreType.DMA((2, f // C)),
            pltpu.SemaphoreType.DMA((f // C,)),
        ],
        compiler_params=pltpu.CompilerParams(
            dimension_semantics=("parallel", "arbitrary")),
    )(mask3, w2, wexp2, wb2, bias2, x)
    return out
